# baseline (speedup 1.0000x reference)
import jax
import jax.numpy as jnp
from jax.experimental import pallas as pl

NUM_EMBEDDINGS = 1024
EMBEDDING_DIM = 64
ROWS_PER_BLOCK = 1024


def _vq_block_kernel(z_ref, e_ref, zq_ref, idx_ref):
    z = z_ref[...]
    e = e_ref[...]
    zsq = jnp.sum(z * z, axis=1, keepdims=True)
    esq = jnp.sum(e * e, axis=1)
    mm2 = jax.lax.dot_general(
        z * (-2.0), e, (((1,), (1,)), ((), ())),
        preferred_element_type=jnp.float32)
    dist = (zsq + esq[None, :]) + mm2
    iota = jax.lax.broadcasted_iota(jnp.int32, dist.shape, 1)
    mins = jnp.min(dist, axis=1, keepdims=True)
    delta = (jax.lax.bitcast_convert_type(dist, jnp.int32)
             - jax.lax.bitcast_convert_type(mins, jnp.int32))
    packed = ((jnp.minimum(delta, (1 << 20) - 1) << 10) | iota) + (1 << 23)
    packed_f = jax.lax.bitcast_convert_type(packed, jnp.float32)
    idx = (jax.lax.bitcast_convert_type(jnp.min(packed_f, axis=1), jnp.int32)
           & (NUM_EMBEDDINGS - 1))
    idx_ref[...] = idx[None, None, :]
    onehot = (iota == idx[:, None]).astype(jnp.float32)
    zq_ref[...] = jax.lax.dot_general(
        onehot, e, (((1,), (0,)), ((), ())),
        preferred_element_type=jnp.float32)


def kernel(z_e, embedding_weight):
    b, c, h, w = z_e.shape
    n = b * h * w
    z_flat = jnp.transpose(z_e, (0, 2, 3, 1)).reshape(n, c)
    nblk = n // ROWS_PER_BLOCK
    zq_flat, idx = pl.pallas_call(
        _vq_block_kernel,
        grid=(nblk,),
        in_specs=[
            pl.BlockSpec((ROWS_PER_BLOCK, c), lambda i: (i, 0)),
            pl.BlockSpec((NUM_EMBEDDINGS, c), lambda i: (0, 0)),
        ],
        out_specs=[
            pl.BlockSpec((ROWS_PER_BLOCK, c), lambda i: (i, 0)),
            pl.BlockSpec((1, 1, ROWS_PER_BLOCK), lambda i: (i, 0, 0)),
        ],
        out_shape=[
            jax.ShapeDtypeStruct((n, c), jnp.float32),
            jax.ShapeDtypeStruct((nblk, 1, ROWS_PER_BLOCK), jnp.int32),
        ],
    )(z_flat, embedding_weight)
    return zq_flat.reshape(z_e.shape), idx.reshape(n)


# speedup vs baseline: 1.1486x; 1.1486x over previous
import jax
import jax.numpy as jnp
from jax.experimental import pallas as pl

NUM_EMBEDDINGS = 1024
EMBEDDING_DIM = 64
ROWS_PER_BLOCK = 2048


def _vq_block_kernel(z_ref, e_ref, zq_ref, idx_ref):
    z = z_ref[...]
    e = e_ref[...]
    k = e.shape[0]
    zsq = jnp.sum(z * z, axis=1, keepdims=True)
    esq = jnp.sum(e * e, axis=1)
    mm2 = jax.lax.dot_general(
        z * (-2.0), e, (((1,), (1,)), ((), ())),
        preferred_element_type=jnp.float32)
    dist = (zsq + esq[None, :]) + mm2
    iota_f = jax.lax.broadcasted_iota(jnp.int32, (1, k), 1).astype(jnp.float32)
    mins = jnp.min(dist, axis=1, keepdims=True)
    masked = jnp.where(dist == mins, iota_f, jnp.float32(k))
    idx_f = jnp.min(masked, axis=1, keepdims=True)
    idx_ref[...] = idx_f.astype(jnp.int32)[:, 0][None, None, :]
    onehot = (iota_f == idx_f).astype(jnp.float32)
    zq_ref[...] = jax.lax.dot_general(
        onehot, e, (((1,), (0,)), ((), ())),
        preferred_element_type=jnp.float32)


def kernel(z_e, embedding_weight):
    b, c, h, w = z_e.shape
    n = b * h * w
    z_flat = jnp.transpose(z_e, (0, 2, 3, 1)).reshape(n, c)
    nblk = n // ROWS_PER_BLOCK
    zq_flat, idx = pl.pallas_call(
        _vq_block_kernel,
        grid=(nblk,),
        in_specs=[
            pl.BlockSpec((ROWS_PER_BLOCK, c), lambda i: (i, 0)),
            pl.BlockSpec((NUM_EMBEDDINGS, c), lambda i: (0, 0)),
        ],
        out_specs=[
            pl.BlockSpec((ROWS_PER_BLOCK, c), lambda i: (i, 0)),
            pl.BlockSpec((1, 1, ROWS_PER_BLOCK), lambda i: (i, 0, 0)),
        ],
        out_shape=[
            jax.ShapeDtypeStruct((n, c), jnp.float32),
            jax.ShapeDtypeStruct((nblk, 1, ROWS_PER_BLOCK), jnp.int32),
        ],
    )(z_flat, embedding_weight)
    return zq_flat.reshape(z_e.shape), idx.reshape(n)


# speedup vs baseline: 1.1859x; 1.0325x over previous
import jax
import jax.numpy as jnp
from jax.experimental import pallas as pl

NUM_EMBEDDINGS = 1024
EMBEDDING_DIM = 64
ROWS_PER_BLOCK = 2048


def _vq_block_kernel(z_ref, e_ref, zq_ref, idx_ref):
    z = z_ref[...]
    e = e_ref[...]
    k = e.shape[0]
    zsq = jnp.sum(z * z, axis=1, keepdims=True)
    esq = jnp.sum(e * e, axis=1)
    mm2 = jax.lax.dot_general(
        z * (-2.0), e, (((1,), (1,)), ((), ())),
        preferred_element_type=jnp.float32)
    dist = (zsq + esq[None, :]) + mm2
    iota_f = jax.lax.broadcasted_iota(jnp.int32, (1, k), 1).astype(jnp.float32)
    mins = jnp.min(dist, axis=1, keepdims=True)
    masked = jnp.where(dist == mins, iota_f, jnp.float32(k))
    idx_f = jnp.min(masked, axis=1, keepdims=True)
    idx_ref[...] = idx_f.astype(jnp.int32)
    onehot = (iota_f == idx_f).astype(jnp.float32)
    zq_ref[...] = jax.lax.dot_general(
        onehot, e, (((1,), (0,)), ((), ())),
        preferred_element_type=jnp.float32)


def kernel(z_e, embedding_weight):
    b, c, h, w = z_e.shape
    n = b * h * w
    z_flat = jnp.transpose(z_e, (0, 2, 3, 1)).reshape(n, c)
    nblk = n // ROWS_PER_BLOCK
    zq_flat, idx = pl.pallas_call(
        _vq_block_kernel,
        grid=(nblk,),
        in_specs=[
            pl.BlockSpec((ROWS_PER_BLOCK, c), lambda i: (i, 0)),
            pl.BlockSpec((NUM_EMBEDDINGS, c), lambda i: (0, 0)),
        ],
        out_specs=[
            pl.BlockSpec((ROWS_PER_BLOCK, c), lambda i: (i, 0)),
            pl.BlockSpec((ROWS_PER_BLOCK, 1), lambda i: (i, 0)),
        ],
        out_shape=[
            jax.ShapeDtypeStruct((n, c), jnp.float32),
            jax.ShapeDtypeStruct((n, 1), jnp.int32),
        ],
    )(z_flat, embedding_weight)
    return zq_flat.reshape(z_e.shape), idx.reshape(n)
